# baseline (speedup 1.0000x reference)
import functools

import jax
import jax.numpy as jnp
from jax import lax
from jax.experimental import pallas as pl
from jax.experimental.pallas import tpu as pltpu
from jax.experimental.pallas import tpu_sc as plsc

_N, _E, _D, _G, _C = 10000, 320000, 128, 64, 10
_NW = 32
_EP = _E
_EPT = _EP // _NW
_NCH, _CH = 80, 125
_NP = 10240
_RPT = _NP // 16
_B = 1000


def _sc_agg(h, es, ed, zrows):
    mesh = plsc.VectorSubcoreMesh(core_axis_name="c", subcore_axis_name="s")

    @functools.partial(
        pl.kernel,
        mesh=mesh,
        out_type=(
            jax.ShapeDtypeStruct((_NP, _D), jnp.float32),
            jax.ShapeDtypeStruct((_NP, _D), jnp.float32),
        ),
        scratch_types=[
            pltpu.VMEM((4, _CH), jnp.int32),
            pltpu.VMEM((4, _CH), jnp.int32),
            pltpu.VMEM((2, _CH, _D), jnp.float32),
            pltpu.VMEM_SHARED((_NP, _D), jnp.float32),
            pltpu.SemaphoreType.DMA,
            pltpu.SemaphoreType.DMA,
            pltpu.SemaphoreType.DMA,
            pltpu.SemaphoreType.DMA,
            pltpu.SemaphoreType.DMA,
            pltpu.SemaphoreType.DMA,
            pltpu.SemaphoreType.DMA,
            pltpu.SemaphoreType.DMA,
            pltpu.SemaphoreType.DMA,
            pltpu.SemaphoreType.DMA,
            pltpu.SemaphoreType.DMA,
            pltpu.SemaphoreType.DMA,
        ],
    )
    def agg(h_hbm, es_hbm, ed_hbm, z_hbm, out0, out1, isrc, idst, rows, acc,
            i0, i1, i2, i3, j0, j1, j2, j3, ga, gb, sa, sb):
        isem = (i0, i1, i2, i3)
        jsem = (j0, j1, j2, j3)
        gsem = (ga, gb)
        ssem = (sa, sb)
        cid = lax.axis_index("c")
        sid = lax.axis_index("s")
        tid = cid * 16 + sid
        pltpu.sync_copy(z_hbm, acc.at[pl.ds(sid * _RPT, _RPT)])

        def istart(j, s):
            pltpu.async_copy(es_hbm.at[tid, j], isrc.at[s], isem[s])
            pltpu.async_copy(ed_hbm.at[tid, j], idst.at[s], jsem[s])

        def iwait(s):
            pltpu.make_async_copy(es_hbm.at[tid, 0], isrc.at[s], isem[s]).wait()
            pltpu.make_async_copy(ed_hbm.at[tid, 0], idst.at[s], jsem[s]).wait()

        def gstart(slot, b):
            pltpu.async_copy(h_hbm.at[isrc.at[slot]], rows.at[b], gsem[b])

        def gwait(b):
            pltpu.make_async_copy(h_hbm.at[isrc.at[0]], rows.at[b],
                                  gsem[b]).wait()

        def scat(slot, b):
            pltpu.async_copy(rows.at[b], acc.at[idst.at[slot]], ssem[0],
                             add=True)
            pltpu.make_async_copy(rows.at[b], acc.at[idst.at[slot]],
                                  ssem[0]).wait()

        istart(0, 0)
        istart(1, 1)
        iwait(0)
        gstart(0, 0)
        iwait(1)
        gstart(1, 1)
        plsc.subcore_barrier()

        def body(g, carry):
            for b in range(4):
                j = 4 * g + b
                istart(j + 2, (b + 2) % 4)
                gwait(b % 2)
                scat(b, b % 2)
                iwait((b + 2) % 4)
                gstart((b + 2) % 4, b % 2)
            return carry

        lax.fori_loop(0, (_NCH - 4) // 4, body, 0)

        for j in range(_NCH - 4, _NCH):
            b = j % 4
            if j + 2 < _NCH:
                istart(j + 2, (b + 2) % 4)
            gwait(b % 2)
            scat(b, b % 2)
            if j + 2 < _NCH:
                iwait((b + 2) % 4)
                gstart((b + 2) % 4, b % 2)
        plsc.subcore_barrier()

        @pl.when(cid == 0)
        def _():
            pltpu.sync_copy(acc.at[pl.ds(sid * _RPT, _RPT)],
                            out0.at[pl.ds(sid * _RPT, _RPT)])

        @pl.when(cid == 1)
        def _():
            pltpu.sync_copy(acc.at[pl.ds(sid * _RPT, _RPT)],
                            out1.at[pl.ds(sid * _RPT, _RPT)])

    return agg(h, es, ed, zrows)


def _mlp_body(x_r, p0_r, p1_r, w1_r, b1_r, w2_r, b2_r):
    hin = x_r[...] + p0_r[...] + p1_r[...]
    t = jnp.dot(hin, w1_r[...], preferred_element_type=jnp.float32) + b1_r[...]
    t = jnp.maximum(t, 0.0)
    t = jnp.dot(t, w2_r[...], preferred_element_type=jnp.float32) + b2_r[...]
    return jnp.maximum(t, 0.0)


def _mlp(x, p0, p1, W1, b1, W2, b2):
    def body(x_r, p0_r, p1_r, w1_r, b1_r, w2_r, b2_r, h_r):
        h_r[...] = _mlp_body(x_r, p0_r, p1_r, w1_r, b1_r, w2_r, b2_r)

    row = pl.BlockSpec((_B, _D), lambda i: (i, 0))
    full = pl.BlockSpec((_D, _D), lambda i: (0, 0))
    bias = pl.BlockSpec((1, _D), lambda i: (0, 0))
    return pl.pallas_call(
        body,
        grid=(_N // _B,),
        in_specs=[row, row, row, full, bias, full, bias],
        out_specs=row,
        out_shape=jax.ShapeDtypeStruct((_N, _D), jnp.float32),
    )(x, p0, p1, W1, b1.reshape(1, _D), W2, b2.reshape(1, _D))


def _mlp_readout(x, p0, p1, W1, b1, W2, b2, batch3):
    def body(x_r, p0_r, p1_r, w1_r, b1_r, w2_r, b2_r, bt_r,
             h_r, s_r, mx_r, cnt_r):
        h = _mlp_body(x_r, p0_r, p1_r, w1_r, b1_r, w2_r, b2_r)
        h_r[...] = h

        @pl.when(pl.program_id(0) == 0)
        def _():
            s_r[...] = jnp.zeros_like(s_r)
            mx_r[...] = jnp.zeros_like(mx_r)
            cnt_r[...] = jnp.zeros_like(cnt_r)

        bvec = bt_r[0, 0, :]
        mask = jnp.where(bvec[:, None] == lax.broadcasted_iota(jnp.int32, (1, _G), 1),
                         1.0, 0.0)
        s_r[...] += lax.dot_general(mask, h, (((0,), (0,)), ((), ())),
                                    preferred_element_type=jnp.float32)
        cnt_r[...] += jnp.broadcast_to(jnp.sum(mask, axis=0)[:, None], (_G, _D))

        glo = jnp.min(bvec)
        ghi = jnp.max(bvec)

        def mbody(g, carry):
            m = jnp.where(bvec == g, 1.0, 0.0)
            contrib = jnp.max(h * m[:, None], axis=0, keepdims=True)
            mx_r[pl.ds(g, 1), :] = jnp.maximum(mx_r[pl.ds(g, 1), :], contrib)
            return carry

        lax.fori_loop(glo, ghi + 1, mbody, 0)

    row = pl.BlockSpec((_B, _D), lambda i: (i, 0))
    full = pl.BlockSpec((_D, _D), lambda i: (0, 0))
    bias = pl.BlockSpec((1, _D), lambda i: (0, 0))
    gblk = pl.BlockSpec((_G, _D), lambda i: (0, 0))
    bblk = pl.BlockSpec((1, 1, _B), lambda i: (i, 0, 0))
    return pl.pallas_call(
        body,
        grid=(_N // _B,),
        in_specs=[row, row, row, full, bias, full, bias, bblk],
        out_specs=[row, gblk, gblk, gblk],
        out_shape=[
            jax.ShapeDtypeStruct((_N, _D), jnp.float32),
            jax.ShapeDtypeStruct((_G, _D), jnp.float32),
            jax.ShapeDtypeStruct((_G, _D), jnp.float32),
            jax.ShapeDtypeStruct((_G, _D), jnp.float32),
        ],
    )(x, p0, p1, W1, b1.reshape(1, _D), W2, b2.reshape(1, _D), batch3)


def _head(s2, mx2, s3, mx3, s4, mx4, cnt, Wl1, bl1, Wl2, bl2):
    def body(s2_r, mx2_r, s3_r, mx3_r, s4_r, mx4_r, cnt_r,
             w1_r, b1_r, w2_r, b2_r, o_r):
        sm = s2_r[...] + s3_r[...] + s4_r[...]
        mxs = mx2_r[...] + mx3_r[...] + mx4_r[...]
        mean = sm / jnp.maximum(cnt_r[...], 1.0)
        g = jnp.concatenate([mean, mxs, sm], axis=1)
        z = jnp.dot(g, w1_r[...], preferred_element_type=jnp.float32) + b1_r[...]
        z = jnp.maximum(z, 0.0)
        lg = jnp.dot(z, w2_r[...], preferred_element_type=jnp.float32) + b2_r[...]
        m = jnp.max(lg, axis=1, keepdims=True)
        lse = jnp.log(jnp.sum(jnp.exp(lg - m), axis=1, keepdims=True)) + m
        o_r[...] = lg - lse

    return pl.pallas_call(
        body,
        out_shape=jax.ShapeDtypeStruct((_G, _C), jnp.float32),
    )(s2, mx2, s3, mx3, s4, mx4, cnt,
      Wl1, bl1.reshape(1, _D), Wl2, bl2.reshape(1, _C))


def kernel(x, edge_index, batch, W1a, b1a, W2a, b2a, W1b, b1b, W2b, b2b,
           Wl1, bl1, Wl2, bl2):
    npad = _EP - _E
    pad_src = jnp.arange(npad, dtype=jnp.int32) % _N
    pad_dst = _N + jnp.arange(npad, dtype=jnp.int32) % (_NP - _N)
    es = jnp.concatenate([edge_index[0], pad_src]).reshape(_NW, _NCH, _CH)
    ed = jnp.concatenate([edge_index[1], pad_dst]).reshape(_NW, _NCH, _CH)
    zrows = jnp.zeros((_RPT, _D), jnp.float32)
    batch3 = batch.reshape(_N // _B, 1, _B)

    p0, p1 = _sc_agg(x, es, ed, zrows)
    h1 = _mlp(x, p0, p1, W1a, b1a, W2a, b2a)
    p0, p1 = _sc_agg(h1, es, ed, zrows)
    h2, s2, mx2, cnt = _mlp_readout(h1, p0, p1, W1b, b1b, W2b, b2b, batch3)
    p0, p1 = _sc_agg(h2, es, ed, zrows)
    h3, s3, mx3, _ = _mlp_readout(h2, p0, p1, W1b, b1b, W2b, b2b, batch3)
    p0, p1 = _sc_agg(h3, es, ed, zrows)
    h4, s4, mx4, _ = _mlp_readout(h3, p0, p1, W1b, b1b, W2b, b2b, batch3)

    return _head(s2, mx2, s3, mx3, s4, mx4, cnt, Wl1, bl1, Wl2, bl2)


# speedup vs baseline: 1.1018x; 1.1018x over previous
import functools

import jax
import jax.numpy as jnp
from jax import lax
from jax.experimental import pallas as pl
from jax.experimental.pallas import tpu as pltpu
from jax.experimental.pallas import tpu_sc as plsc

_N, _E, _D, _G, _C = 10000, 320000, 128, 64, 10
_NW = 32
_EP = _E
_EPT = _EP // _NW
_NCH, _CH = 80, 125
_NP = 10240
_RPT = _NP // 16
_B = 1000


def _sc_agg(h, e3, zrows):
    mesh = plsc.VectorSubcoreMesh(core_axis_name="c", subcore_axis_name="s")

    @functools.partial(
        pl.kernel,
        mesh=mesh,
        out_type=(
            jax.ShapeDtypeStruct((_NP, _D), jnp.float32),
            jax.ShapeDtypeStruct((_NP, _D), jnp.float32),
        ),
        scratch_types=[
            pltpu.VMEM((4, 2, _CH), jnp.int32),
            pltpu.VMEM((2, _CH, _D), jnp.float32),
            pltpu.VMEM_SHARED((_NP, _D), jnp.float32),
            pltpu.SemaphoreType.DMA,
            pltpu.SemaphoreType.DMA,
            pltpu.SemaphoreType.DMA,
            pltpu.SemaphoreType.DMA,
            pltpu.SemaphoreType.DMA,
            pltpu.SemaphoreType.DMA,
            pltpu.SemaphoreType.DMA,
        ],
    )
    def agg(h_hbm, e_hbm, z_hbm, out0, out1, ibuf, rows, acc,
            i0, i1, i2, i3, ga, gb, sa):
        isem = (i0, i1, i2, i3)
        gsem = (ga, gb)
        ssem = (sa,)
        cid = lax.axis_index("c")
        sid = lax.axis_index("s")
        tid = cid * 16 + sid
        def istart(j, s):
            pltpu.async_copy(e_hbm.at[tid, j], ibuf.at[s], isem[s])

        def iwait(s):
            pltpu.make_async_copy(e_hbm.at[tid, 0], ibuf.at[s], isem[s]).wait()

        def gstart(slot, b):
            pltpu.async_copy(h_hbm.at[ibuf.at[slot, 0]], rows.at[b], gsem[b])

        def gwait(b):
            pltpu.make_async_copy(h_hbm.at[ibuf.at[0, 0]], rows.at[b],
                                  gsem[b]).wait()

        def scat(slot, b):
            pltpu.sync_copy(rows.at[b], acc.at[ibuf.at[slot, 1]], add=True)

        istart(0, 0)
        istart(1, 1)
        iwait(0)
        gstart(0, 0)
        iwait(1)
        gstart(1, 1)

        @pl.when(cid == 0)
        def _():
            @pl.when(sid < 15)
            def _():
                pltpu.sync_copy(h_hbm.at[pl.ds(sid * _RPT, _RPT)],
                                acc.at[pl.ds(sid * _RPT, _RPT)])

            @pl.when(sid == 15)
            def _():
                pltpu.sync_copy(h_hbm.at[pl.ds(15 * _RPT, _N - 15 * _RPT)],
                                acc.at[pl.ds(15 * _RPT, _N - 15 * _RPT)])
                pltpu.sync_copy(z_hbm.at[pl.ds(0, _NP - _N)],
                                acc.at[pl.ds(_N, _NP - _N)])

        @pl.when(cid == 1)
        def _():
            pltpu.sync_copy(z_hbm, acc.at[pl.ds(sid * _RPT, _RPT)])

        plsc.subcore_barrier()

        def body(g, carry):
            for b in range(4):
                j = 4 * g + b
                istart(j + 2, (b + 2) % 4)
                gwait(b % 2)
                scat(b, b % 2)
                iwait((b + 2) % 4)
                gstart((b + 2) % 4, b % 2)
            return carry

        lax.fori_loop(0, (_NCH - 4) // 4, body, 0)

        for j in range(_NCH - 4, _NCH):
            b = j % 4
            if j + 2 < _NCH:
                istart(j + 2, (b + 2) % 4)
            gwait(b % 2)
            scat(b, b % 2)
            if j + 2 < _NCH:
                iwait((b + 2) % 4)
                gstart((b + 2) % 4, b % 2)
        plsc.subcore_barrier()

        @pl.when(cid == 0)
        def _():
            pltpu.sync_copy(acc.at[pl.ds(sid * _RPT, _RPT)],
                            out0.at[pl.ds(sid * _RPT, _RPT)])

        @pl.when(cid == 1)
        def _():
            pltpu.sync_copy(acc.at[pl.ds(sid * _RPT, _RPT)],
                            out1.at[pl.ds(sid * _RPT, _RPT)])

    return agg(h, e3, zrows)


def _mlp_body(p0_r, p1_r, w1_r, b1_r, w2_r, b2_r):
    hin = p0_r[...] + p1_r[...]
    t = jnp.dot(hin, w1_r[...], preferred_element_type=jnp.float32) + b1_r[...]
    t = jnp.maximum(t, 0.0)
    t = jnp.dot(t, w2_r[...], preferred_element_type=jnp.float32) + b2_r[...]
    return jnp.maximum(t, 0.0)


def _mlp(p0, p1, W1, b1, W2, b2):
    def body(p0_r, p1_r, w1_r, b1_r, w2_r, b2_r, h_r):
        h_r[...] = _mlp_body(p0_r, p1_r, w1_r, b1_r, w2_r, b2_r)

    row = pl.BlockSpec((_B, _D), lambda i: (i, 0))
    full = pl.BlockSpec((_D, _D), lambda i: (0, 0))
    bias = pl.BlockSpec((1, _D), lambda i: (0, 0))
    return pl.pallas_call(
        body,
        grid=(_N // _B,),
        in_specs=[row, row, full, bias, full, bias],
        out_specs=row,
        out_shape=jax.ShapeDtypeStruct((_N, _D), jnp.float32),
    )(p0, p1, W1, b1.reshape(1, _D), W2, b2.reshape(1, _D))


def _readout(h, batch3):
    def body(h_r, bt_r, s_r, mx_r, cnt_r):
        h = h_r[...]

        @pl.when(pl.program_id(0) == 0)
        def _():
            s_r[...] = jnp.zeros_like(s_r)
            mx_r[...] = jnp.zeros_like(mx_r)
            cnt_r[...] = jnp.zeros_like(cnt_r)

        bvec = bt_r[0, 0, :]
        mask = jnp.where(bvec[:, None] == lax.broadcasted_iota(jnp.int32, (1, _G), 1),
                         1.0, 0.0)
        s_r[...] += lax.dot_general(mask, h, (((0,), (0,)), ((), ())),
                                    preferred_element_type=jnp.float32)
        cnt_r[...] += jnp.broadcast_to(jnp.sum(mask, axis=0)[:, None], (_G, _D))

        glo = jnp.min(bvec)
        ghi = jnp.max(bvec)

        def mbody(g, carry):
            m = jnp.where(bvec == g, 1.0, 0.0)
            contrib = jnp.max(h * m[:, None], axis=0, keepdims=True)
            mx_r[pl.ds(g, 1), :] = jnp.maximum(mx_r[pl.ds(g, 1), :], contrib)
            return carry

        lax.fori_loop(glo, ghi + 1, mbody, 0)

    row = pl.BlockSpec((_B, _D), lambda i: (i, 0))
    gblk = pl.BlockSpec((_G, _D), lambda i: (0, 0))
    bblk = pl.BlockSpec((1, 1, _B), lambda i: (i, 0, 0))
    return pl.pallas_call(
        body,
        grid=(_N // _B,),
        in_specs=[row, bblk],
        out_specs=[gblk, gblk, gblk],
        out_shape=[
            jax.ShapeDtypeStruct((_G, _D), jnp.float32),
            jax.ShapeDtypeStruct((_G, _D), jnp.float32),
            jax.ShapeDtypeStruct((_G, _D), jnp.float32),
        ],
    )(h, batch3)


def _head(s2, mx2, s3, mx3, s4, mx4, cnt, Wl1, bl1, Wl2, bl2):
    def body(s2_r, mx2_r, s3_r, mx3_r, s4_r, mx4_r, cnt_r,
             w1_r, b1_r, w2_r, b2_r, o_r):
        sm = s2_r[...] + s3_r[...] + s4_r[...]
        mxs = mx2_r[...] + mx3_r[...] + mx4_r[...]
        mean = sm / jnp.maximum(cnt_r[...], 1.0)
        g = jnp.concatenate([mean, mxs, sm], axis=1)
        z = jnp.dot(g, w1_r[...], preferred_element_type=jnp.float32) + b1_r[...]
        z = jnp.maximum(z, 0.0)
        lg = jnp.dot(z, w2_r[...], preferred_element_type=jnp.float32) + b2_r[...]
        m = jnp.max(lg, axis=1, keepdims=True)
        lse = jnp.log(jnp.sum(jnp.exp(lg - m), axis=1, keepdims=True)) + m
        o_r[...] = lg - lse

    return pl.pallas_call(
        body,
        out_shape=jax.ShapeDtypeStruct((_G, _C), jnp.float32),
    )(s2, mx2, s3, mx3, s4, mx4, cnt,
      Wl1, bl1.reshape(1, _D), Wl2, bl2.reshape(1, _C))


def kernel(x, edge_index, batch, W1a, b1a, W2a, b2a, W1b, b1b, W2b, b2b,
           Wl1, bl1, Wl2, bl2):
    e3 = edge_index.reshape(2, _NW, _NCH, _CH).transpose(1, 2, 0, 3)
    zrows = jnp.zeros((_RPT, _D), jnp.float32)
    batch3 = batch.reshape(_N // _B, 1, _B)

    p0, p1 = _sc_agg(x, e3, zrows)
    h1 = _mlp(p0, p1, W1a, b1a, W2a, b2a)
    p0, p1 = _sc_agg(h1, e3, zrows)
    h2 = _mlp(p0, p1, W1b, b1b, W2b, b2b)
    p0, p1 = _sc_agg(h2, e3, zrows)
    s2, mx2, cnt = _readout(h2, batch3)
    h3 = _mlp(p0, p1, W1b, b1b, W2b, b2b)
    p0, p1 = _sc_agg(h3, e3, zrows)
    s3, mx3, _ = _readout(h3, batch3)
    h4 = _mlp(p0, p1, W1b, b1b, W2b, b2b)
    s4, mx4, _ = _readout(h4, batch3)

    return _head(s2, mx2, s3, mx3, s4, mx4, cnt, Wl1, bl1, Wl2, bl2)
